# baseline (speedup 1.0000x reference)
import functools

import jax
import jax.numpy as jnp
from jax import lax
from jax.experimental import pallas as pl
from jax.experimental.pallas import tpu as pltpu
from jax.experimental.pallas import tpu_sc as plsc

NC, NS, LANES = 2, 16, 16
NW = NC * NS
EB = 128
F0 = 0.50


def _make_seg_sum(R, D, NB0, NB1_lo, rem1, B0, with_cnt):
  rpt = R // NS
  NB1_hi = NB1_lo + (1 if rem1 else 0)
  NBmax = max(NB0, NB1_hi)

  out_type = [jax.ShapeDtypeStruct((NC, R, D), jnp.float32)]
  scratch = [
      pltpu.VMEM_SHARED((R, D), jnp.float32),
      pltpu.VMEM((NBmax, EB), jnp.int32),
      pltpu.VMEM((NBmax, EB), jnp.int32),
      pltpu.VMEM((EB, D), jnp.float32),
      pltpu.VMEM((EB, D), jnp.float32),
      pltpu.SemaphoreType.DMA,
      pltpu.SemaphoreType.DMA,
      pltpu.SemaphoreType.DMA,
      pltpu.SemaphoreType.DMA,
  ]
  if with_cnt:
    out_type.append(jax.ShapeDtypeStruct((NC, R, LANES), jnp.float32))
    scratch += [
        pltpu.VMEM_SHARED((R, LANES), jnp.float32),
        pltpu.VMEM((EB, LANES), jnp.float32),
        pltpu.SemaphoreType.DMA,
    ]

  mesh = plsc.VectorSubcoreMesh(core_axis_name="c", subcore_axis_name="s")

  @functools.partial(pl.kernel, out_type=tuple(out_type), mesh=mesh,
                     scratch_types=tuple(scratch),
                     compiler_params=pltpu.CompilerParams(
                         use_tc_tiling_on_sc=False))
  def seg(*refs):
    if with_cnt:
      (y_hbm, ei_hbm, zD_hbm, zc_hbm, ones_hbm,
       outS, outC, acc, src_v, dst_v, gb0, gb1, gs0, gs1, ss0, ss1,
       cnt_acc, ones_v, osem) = refs
    else:
      (y_hbm, ei_hbm, zD_hbm,
       outS, acc, src_v, dst_v, gb0, gb1, gs0, gs1, ss0, ss1) = refs
    gbuf = (gb0, gb1)
    gsem = (gs0, gs1)
    ssem = (ss0, ss1)

    c = lax.axis_index("c")
    s = lax.axis_index("s")
    r0 = s * rpt
    start = jnp.where(c == 0, s * NB0,
                      B0 + s * NB1_lo + jnp.minimum(s, rem1))
    nb = jnp.where(c == 0, NB0, NB1_lo + (s < rem1))

    pltpu.sync_copy(zD_hbm.at[pl.ds(r0, rpt)], acc.at[pl.ds(r0, rpt)])
    if with_cnt:
      pltpu.sync_copy(zc_hbm.at[pl.ds(r0, rpt)], cnt_acc.at[pl.ds(r0, rpt)])
      pltpu.sync_copy(ones_hbm, ones_v)
    @pl.when(c == 0)
    def _():
      pltpu.sync_copy(ei_hbm.at[0, pl.ds(start, NB0)], src_v.at[pl.ds(0, NB0)])
      pltpu.sync_copy(ei_hbm.at[1, pl.ds(start, NB0)], dst_v.at[pl.ds(0, NB0)])
    @pl.when(c != 0)
    def _():
      pltpu.sync_copy(ei_hbm.at[0, pl.ds(start, NB1_lo)],
                      src_v.at[pl.ds(0, NB1_lo)])
      pltpu.sync_copy(ei_hbm.at[1, pl.ds(start, NB1_lo)],
                      dst_v.at[pl.ds(0, NB1_lo)])
    if rem1:
      @pl.when(jnp.logical_and(c != 0, s < rem1))
      def _():
        pltpu.sync_copy(ei_hbm.at[0, pl.ds(start + NB1_lo, 1)],
                        src_v.at[pl.ds(NB1_lo, 1)])
        pltpu.sync_copy(ei_hbm.at[1, pl.ds(start + NB1_lo, 1)],
                        dst_v.at[pl.ds(NB1_lo, 1)])
    plsc.subcore_barrier()

    def start_gather(j, b):
      pltpu.async_copy(y_hbm.at[src_v.at[j]], gbuf[b], gsem[b])

    def wait_gather(j, b):
      pltpu.make_async_copy(y_hbm.at[src_v.at[j]], gbuf[b], gsem[b]).wait()

    def scatter(j, b):
      if with_cnt:
        pltpu.async_copy(ones_v, cnt_acc.at[dst_v.at[j]], osem, add=True)
      pltpu.async_copy(gbuf[b], acc.at[dst_v.at[j]], ssem[b], add=True)

    def wait_scatter(j, b):
      pltpu.make_async_copy(gbuf[b], acc.at[dst_v.at[j]], ssem[b]).wait()
      if with_cnt:
        pltpu.make_async_copy(ones_v, cnt_acc.at[dst_v.at[j]], osem).wait()

    npairs = nb // 2

    @pl.when(npairs > 0)
    def _():
      start_gather(0, 0)
      start_gather(1, 1)

    def pair(t, carry):
      j0 = 2 * t
      wait_gather(j0, 0)
      scatter(j0, 0)
      wait_gather(j0 + 1, 1)
      scatter(j0 + 1, 1)
      wait_scatter(j0, 0)
      wait_scatter(j0 + 1, 1)
      @pl.when(t + 1 < npairs)
      def _():
        start_gather(j0 + 2, 0)
        start_gather(j0 + 3, 1)
      return carry
    lax.fori_loop(0, npairs, pair, 0)

    @pl.when(nb % 2 == 1)
    def _():
      j = nb - 1
      start_gather(j, 0)
      wait_gather(j, 0)
      scatter(j, 0)
      wait_scatter(j, 0)

    plsc.subcore_barrier()
    pltpu.sync_copy(acc.at[pl.ds(r0, rpt)], outS.at[c, pl.ds(r0, rpt)])
    if with_cnt:
      pltpu.sync_copy(cnt_acc.at[pl.ds(r0, rpt)], outC.at[c, pl.ds(r0, rpt)])

  return seg


def _stage_a_body(x_ref, w_ref, y1_ref, r1_ref):
  m = jnp.dot(x_ref[...], w_ref[...], preferred_element_type=jnp.float32)
  h = w_ref.shape[1] // 2
  y1_ref[...] = m[:, :h]
  r1_ref[...] = m[:, h:]


def _stage_b_body(s1p_ref, cntp_ref, r1_ref, b1_ref, w2_ref, y2_ref, r2_ref):
  cnt = cntp_ref[0, :, 0:1] + cntp_ref[1, :, 0:1]
  inv = 1.0 / jnp.maximum(cnt, 1.0)
  s1 = s1p_ref[0] + s1p_ref[1]
  h = jnp.maximum(s1 * inv + r1_ref[...] + b1_ref[...], 0.0)
  ycat2 = jnp.dot(h, w2_ref[...], preferred_element_type=jnp.float32)
  l_w = w2_ref.shape[1] // 2
  y2_ref[...] = ycat2[:, :l_w]
  r2_ref[...] = ycat2[:, l_w:]


def _stage_c_body(s2p_ref, cntp_ref, r2_ref, b2_ref, fc1w_ref, fc1b_ref,
                  fc2w_ref, fc2b_ref, o_ref):
  cnt = cntp_ref[0, :, 0:1] + cntp_ref[1, :, 0:1]
  inv = 1.0 / jnp.maximum(cnt, 1.0)
  z = (s2p_ref[0] + s2p_ref[1]) * inv + r2_ref[...] + b2_ref[...]
  d = jnp.maximum(
      jnp.dot(z, fc1w_ref[...], preferred_element_type=jnp.float32)
      + fc1b_ref[...], 0.0)
  o_ref[...] = (jnp.dot(d, fc2w_ref[...], preferred_element_type=jnp.float32)
                + fc2b_ref[...])


def _row_spec(bm, width):
  return pl.BlockSpec((bm, width), lambda i: (i, 0))


def _full_spec(shape):
  nd = len(shape)
  return pl.BlockSpec(shape, lambda i: (0,) * nd)


def _part_spec(bm, width):
  return pl.BlockSpec((NC, bm, width), lambda i: (0, i, 0))


def kernel(x, edge_index, conv1_weight, conv1_root, conv1_bias,
           conv2_weight, conv2_root, conv2_bias, fc1_w, fc1_b, fc2_w, fc2_b):
  N, IN = x.shape
  E = edge_index.shape[1]
  H = conv1_root.shape[1]
  L = conv2_root.shape[1]

  R = ((N + NS - 1) // NS + 7) // 8 * 8 * NS
  TB = -(-E // EB)
  B0 = min(max(NS * round(TB * F0 / NS), NS), TB - NS)
  NB0 = B0 // NS
  B1 = TB - B0
  NB1_lo, rem1 = divmod(B1, NS)

  if E == TB * EB:
    ei3 = edge_index.reshape(2, TB, EB)
  else:
    pad = jnp.full((2, TB * EB - E), N, jnp.int32)
    ei3 = jnp.concatenate([edge_index, pad], axis=1).reshape(2, TB, EB)
  w1cat = jnp.concatenate([conv1_weight[0], conv1_root], axis=1)
  w2cat = jnp.concatenate([conv2_weight[0], conv2_root], axis=1)
  zH = jnp.zeros((R, H), jnp.float32)
  zL = jnp.zeros((R, L), jnp.float32)
  zc = jnp.zeros((R, LANES), jnp.float32)
  ones_rows = jnp.ones((EB, LANES), jnp.float32)

  bm = R // 4
  grid = (R // bm,)

  y1, r1 = pl.pallas_call(
      _stage_a_body, grid=grid,
      in_specs=[_row_spec(bm, IN), _full_spec((IN, 2 * H))],
      out_specs=[_row_spec(bm, H), _row_spec(bm, H)],
      out_shape=[jax.ShapeDtypeStruct((R, H), jnp.float32),
                 jax.ShapeDtypeStruct((R, H), jnp.float32)],
  )(x, w1cat)

  seg1 = _make_seg_sum(R, H, NB0, NB1_lo, rem1, B0, with_cnt=True)
  s1p, cntp = seg1(y1, ei3, zH, zc, ones_rows)

  y2, r2 = pl.pallas_call(
      _stage_b_body, grid=grid,
      in_specs=[_part_spec(bm, H), _part_spec(bm, LANES), _row_spec(bm, H),
                _full_spec((1, H)), _full_spec((H, 2 * L))],
      out_specs=[_row_spec(bm, L), _row_spec(bm, L)],
      out_shape=[jax.ShapeDtypeStruct((R, L), jnp.float32),
                 jax.ShapeDtypeStruct((R, L), jnp.float32)],
  )(s1p, cntp, r1, conv1_bias.reshape(1, H), w2cat)

  seg2 = _make_seg_sum(R, L, NB0, NB1_lo, rem1, B0, with_cnt=False)
  (s2p,) = seg2(y2, ei3, zL)

  bm_c = 2048
  out = pl.pallas_call(
      _stage_c_body, grid=(-(-N // bm_c),),
      in_specs=[_part_spec(bm_c, L), _part_spec(bm_c, LANES),
                _row_spec(bm_c, L),
                _full_spec((1, L)), _full_spec((L, H)), _full_spec((1, H)),
                _full_spec((H, IN)), _full_spec((1, IN))],
      out_specs=_row_spec(bm_c, IN),
      out_shape=jax.ShapeDtypeStruct((N, IN), jnp.float32),
  )(s2p, cntp, r2, conv2_bias.reshape(1, L), fc1_w, fc1_b.reshape(1, H),
    fc2_w, fc2_b.reshape(1, IN))

  return out


# speedup vs baseline: 1.1378x; 1.1378x over previous
import functools

import jax
import jax.numpy as jnp
from jax import lax
from jax.experimental import pallas as pl
from jax.experimental.pallas import tpu as pltpu
from jax.experimental.pallas import tpu_sc as plsc

NC, NS, LANES = 2, 16, 16
NW = NC * NS
EB = 128
F0 = 0.50


def _make_seg_sum(R, D, NB0, NB1_lo, rem1, B0, with_cnt):
  rpt = R // NS
  NB1_hi = NB1_lo + (1 if rem1 else 0)
  NBmax = max(NB0, NB1_hi)

  out_type = [jax.ShapeDtypeStruct((NC, R, D), jnp.float32)]
  scratch = [
      pltpu.VMEM_SHARED((R, D), jnp.float32),
      pltpu.VMEM((NBmax, EB), jnp.int32),
      pltpu.VMEM((NBmax, EB), jnp.int32),
      pltpu.VMEM((EB, D), jnp.float32),
      pltpu.VMEM((EB, D), jnp.float32),
      pltpu.SemaphoreType.DMA,
      pltpu.SemaphoreType.DMA,
      pltpu.SemaphoreType.DMA,
      pltpu.SemaphoreType.DMA,
  ]
  if with_cnt:
    out_type.append(jax.ShapeDtypeStruct((NC, R, LANES), jnp.float32))
    scratch += [
        pltpu.VMEM_SHARED((R, LANES), jnp.float32),
        pltpu.VMEM((EB, LANES), jnp.float32),
        pltpu.SemaphoreType.DMA,
    ]

  mesh = plsc.VectorSubcoreMesh(core_axis_name="c", subcore_axis_name="s")

  @functools.partial(pl.kernel, out_type=tuple(out_type), mesh=mesh,
                     scratch_types=tuple(scratch),
                     compiler_params=pltpu.CompilerParams(
                         use_tc_tiling_on_sc=False))
  def seg(*refs):
    if with_cnt:
      (y_hbm, ei_hbm, zD_hbm, zc_hbm, ones_hbm,
       outS, outC, acc, src_v, dst_v, gb0, gb1, gs0, gs1, ss0, ss1,
       cnt_acc, ones_v, osem) = refs
    else:
      (y_hbm, ei_hbm, zD_hbm,
       outS, acc, src_v, dst_v, gb0, gb1, gs0, gs1, ss0, ss1) = refs
    gbuf = (gb0, gb1)
    gsem = (gs0, gs1)
    ssem = (ss0, ss1)

    c = lax.axis_index("c")
    s = lax.axis_index("s")
    r0 = s * rpt
    start = jnp.where(c == 0, s * NB0,
                      B0 + s * NB1_lo + jnp.minimum(s, rem1))
    nb = jnp.where(c == 0, NB0, NB1_lo + (s < rem1))

    pltpu.sync_copy(zD_hbm.at[pl.ds(r0, rpt)], acc.at[pl.ds(r0, rpt)])
    if with_cnt:
      pltpu.sync_copy(zc_hbm.at[pl.ds(r0, rpt)], cnt_acc.at[pl.ds(r0, rpt)])
      pltpu.sync_copy(ones_hbm, ones_v)
    @pl.when(c == 0)
    def _():
      pltpu.sync_copy(ei_hbm.at[0, pl.ds(start, NB0)], src_v.at[pl.ds(0, NB0)])
      pltpu.sync_copy(ei_hbm.at[1, pl.ds(start, NB0)], dst_v.at[pl.ds(0, NB0)])
    @pl.when(c != 0)
    def _():
      pltpu.sync_copy(ei_hbm.at[0, pl.ds(start, NB1_lo)],
                      src_v.at[pl.ds(0, NB1_lo)])
      pltpu.sync_copy(ei_hbm.at[1, pl.ds(start, NB1_lo)],
                      dst_v.at[pl.ds(0, NB1_lo)])
    if rem1:
      @pl.when(jnp.logical_and(c != 0, s < rem1))
      def _():
        pltpu.sync_copy(ei_hbm.at[0, pl.ds(start + NB1_lo, 1)],
                        src_v.at[pl.ds(NB1_lo, 1)])
        pltpu.sync_copy(ei_hbm.at[1, pl.ds(start + NB1_lo, 1)],
                        dst_v.at[pl.ds(NB1_lo, 1)])
    plsc.subcore_barrier()

    def start_gather(j, b):
      pltpu.async_copy(y_hbm.at[src_v.at[j]], gbuf[b], gsem[b])

    def wait_gather(j, b):
      pltpu.make_async_copy(y_hbm.at[src_v.at[j]], gbuf[b], gsem[b]).wait()

    def scatter(j, b):
      if with_cnt:
        pltpu.async_copy(ones_v, cnt_acc.at[dst_v.at[j]], osem, add=True)
      pltpu.async_copy(gbuf[b], acc.at[dst_v.at[j]], ssem[b], add=True)

    def wait_scatter(j, b):
      pltpu.make_async_copy(gbuf[b], acc.at[dst_v.at[j]], ssem[b]).wait()
      if with_cnt:
        pltpu.make_async_copy(ones_v, cnt_acc.at[dst_v.at[j]], osem).wait()

    npairs = nb // 2

    @pl.when(npairs > 0)
    def _():
      start_gather(0, 0)

    def pair(t, carry):
      j0 = 2 * t
      start_gather(j0 + 1, 1)
      wait_gather(j0, 0)
      scatter(j0, 0)
      wait_scatter(j0, 0)
      @pl.when(t + 1 < npairs)
      def _():
        start_gather(j0 + 2, 0)
      wait_gather(j0 + 1, 1)
      scatter(j0 + 1, 1)
      wait_scatter(j0 + 1, 1)
      return carry
    lax.fori_loop(0, npairs, pair, 0)

    @pl.when(nb % 2 == 1)
    def _():
      j = nb - 1
      start_gather(j, 0)
      wait_gather(j, 0)
      scatter(j, 0)
      wait_scatter(j, 0)

    plsc.subcore_barrier()
    pltpu.sync_copy(acc.at[pl.ds(r0, rpt)], outS.at[c, pl.ds(r0, rpt)])
    if with_cnt:
      pltpu.sync_copy(cnt_acc.at[pl.ds(r0, rpt)], outC.at[c, pl.ds(r0, rpt)])

  return seg


def _stage_a_body(x_ref, w_ref, y1_ref, r1_ref):
  m = jnp.dot(x_ref[...], w_ref[...], preferred_element_type=jnp.float32)
  h = w_ref.shape[1] // 2
  y1_ref[...] = m[:, :h]
  r1_ref[...] = m[:, h:]


def _stage_b_body(s1p_ref, cntp_ref, r1_ref, b1_ref, w2_ref, y2_ref, r2_ref):
  cnt = cntp_ref[0, :, 0:1] + cntp_ref[1, :, 0:1]
  inv = 1.0 / jnp.maximum(cnt, 1.0)
  s1 = s1p_ref[0] + s1p_ref[1]
  h = jnp.maximum(s1 * inv + r1_ref[...] + b1_ref[...], 0.0)
  ycat2 = jnp.dot(h, w2_ref[...], preferred_element_type=jnp.float32)
  l_w = w2_ref.shape[1] // 2
  y2_ref[...] = ycat2[:, :l_w]
  r2_ref[...] = ycat2[:, l_w:]


def _stage_c_body(s2p_ref, cntp_ref, r2_ref, b2_ref, fc1w_ref, fc1b_ref,
                  fc2w_ref, fc2b_ref, o_ref):
  cnt = cntp_ref[0, :, 0:1] + cntp_ref[1, :, 0:1]
  inv = 1.0 / jnp.maximum(cnt, 1.0)
  z = (s2p_ref[0] + s2p_ref[1]) * inv + r2_ref[...] + b2_ref[...]
  d = jnp.maximum(
      jnp.dot(z, fc1w_ref[...], preferred_element_type=jnp.float32)
      + fc1b_ref[...], 0.0)
  o_ref[...] = (jnp.dot(d, fc2w_ref[...], preferred_element_type=jnp.float32)
                + fc2b_ref[...])


def _row_spec(bm, width):
  return pl.BlockSpec((bm, width), lambda i: (i, 0))


def _full_spec(shape):
  nd = len(shape)
  return pl.BlockSpec(shape, lambda i: (0,) * nd)


def _part_spec(bm, width):
  return pl.BlockSpec((NC, bm, width), lambda i: (0, i, 0))


def kernel(x, edge_index, conv1_weight, conv1_root, conv1_bias,
           conv2_weight, conv2_root, conv2_bias, fc1_w, fc1_b, fc2_w, fc2_b):
  N, IN = x.shape
  E = edge_index.shape[1]
  H = conv1_root.shape[1]
  L = conv2_root.shape[1]

  R = ((N + NS - 1) // NS + 7) // 8 * 8 * NS
  TB = -(-E // EB)
  B0 = min(max(NS * round(TB * F0 / NS), NS), TB - NS)
  NB0 = B0 // NS
  B1 = TB - B0
  NB1_lo, rem1 = divmod(B1, NS)

  if E == TB * EB:
    ei3 = edge_index.reshape(2, TB, EB)
  else:
    pad = jnp.full((2, TB * EB - E), N, jnp.int32)
    ei3 = jnp.concatenate([edge_index, pad], axis=1).reshape(2, TB, EB)
  w1cat = jnp.concatenate([conv1_weight[0], conv1_root], axis=1)
  w2cat = jnp.concatenate([conv2_weight[0], conv2_root], axis=1)
  zH = jnp.zeros((R, H), jnp.float32)
  zL = jnp.zeros((R, L), jnp.float32)
  zc = jnp.zeros((R, LANES), jnp.float32)
  ones_rows = jnp.ones((EB, LANES), jnp.float32)

  bm = R // 4
  grid = (R // bm,)

  y1, r1 = pl.pallas_call(
      _stage_a_body, grid=grid,
      in_specs=[_row_spec(bm, IN), _full_spec((IN, 2 * H))],
      out_specs=[_row_spec(bm, H), _row_spec(bm, H)],
      out_shape=[jax.ShapeDtypeStruct((R, H), jnp.float32),
                 jax.ShapeDtypeStruct((R, H), jnp.float32)],
  )(x, w1cat)

  seg1 = _make_seg_sum(R, H, NB0, NB1_lo, rem1, B0, with_cnt=True)
  s1p, cntp = seg1(y1, ei3, zH, zc, ones_rows)

  y2, r2 = pl.pallas_call(
      _stage_b_body, grid=grid,
      in_specs=[_part_spec(bm, H), _part_spec(bm, LANES), _row_spec(bm, H),
                _full_spec((1, H)), _full_spec((H, 2 * L))],
      out_specs=[_row_spec(bm, L), _row_spec(bm, L)],
      out_shape=[jax.ShapeDtypeStruct((R, L), jnp.float32),
                 jax.ShapeDtypeStruct((R, L), jnp.float32)],
  )(s1p, cntp, r1, conv1_bias.reshape(1, H), w2cat)

  seg2 = _make_seg_sum(R, L, NB0, NB1_lo, rem1, B0, with_cnt=False)
  (s2p,) = seg2(y2, ei3, zL)

  bm_c = 2048
  out = pl.pallas_call(
      _stage_c_body, grid=(-(-N // bm_c),),
      in_specs=[_part_spec(bm_c, L), _part_spec(bm_c, LANES),
                _row_spec(bm_c, L),
                _full_spec((1, L)), _full_spec((L, H)), _full_spec((1, H)),
                _full_spec((H, IN)), _full_spec((1, IN))],
      out_specs=_row_spec(bm_c, IN),
      out_shape=jax.ShapeDtypeStruct((N, IN), jnp.float32),
  )(s2p, cntp, r2, conv2_bias.reshape(1, L), fc1_w, fc1_b.reshape(1, H),
    fc2_w, fc2_b.reshape(1, IN))

  return out
